# Initial kernel scaffold; baseline (speedup 1.0000x reference)
import jax
import jax.numpy as jnp
from jax.experimental import pallas as pl


def kernel(x, W, gate_W, As, Bs):
    raise NotImplementedError("write your pallas kernel here")


# speedup vs baseline: 9.4474x; 9.4474x over previous
import functools

import jax
import jax.numpy as jnp
from jax.experimental import pallas as pl

_ALPHA = 16.0
_NT = (((1,), (1,)), ((), ()))


def _moe_lora_kernel(x_ref, w_ref, gate_ref, a_ref, bmat_ref, out_ref, *, E, r):
    xt = x_ref[...]
    base = jax.lax.dot_general(xt, w_ref[...], _NT,
                               preferred_element_type=jnp.float32)
    logits = jax.lax.dot_general(xt, gate_ref[...], _NT,
                                 preferred_element_type=jnp.float32)
    h = jax.lax.dot_general(xt, a_ref[...], _NT,
                            preferred_element_type=jnp.float32)

    T = logits.shape[0]
    eid = jax.lax.broadcasted_iota(jnp.int32, (T, E), 1)
    m1 = jnp.max(logits, axis=1, keepdims=True)
    i1 = jnp.min(jnp.where(logits == m1, eid, E), axis=1, keepdims=True)
    masked = jnp.where(eid == i1, -jnp.inf, logits)
    m2 = jnp.max(masked, axis=1, keepdims=True)
    i2 = jnp.min(jnp.where(masked == m2, eid, E), axis=1, keepdims=True)
    w2 = 1.0 / (1.0 + jnp.exp(m1 - m2))
    w1 = 1.0 - w2

    ke = jax.lax.broadcasted_iota(jnp.int32, (T, E * r), 1) // r
    c = jnp.where(ke == i1, w1, 0.0) + jnp.where(ke == i2, w2, 0.0)
    hw = h * (c * _ALPHA)
    lora = jax.lax.dot_general(hw, bmat_ref[...], _NT,
                               preferred_element_type=jnp.float32)
    out_ref[...] = base + lora


def kernel(x, W, gate_W, As, Bs):
    B, S, H = x.shape
    O = W.shape[0]
    E, r, _ = As.shape
    N = B * S
    xf = x.reshape(N, H)
    A_flat = As.reshape(E * r, H)
    Bmat = jnp.transpose(Bs, (1, 0, 2)).reshape(O, E * r)

    T = 512
    grid = (N // T,)
    out = pl.pallas_call(
        functools.partial(_moe_lora_kernel, E=E, r=r),
        grid=grid,
        in_specs=[
            pl.BlockSpec((T, H), lambda i: (i, 0)),
            pl.BlockSpec((O, H), lambda i: (0, 0)),
            pl.BlockSpec((E, H), lambda i: (0, 0)),
            pl.BlockSpec((E * r, H), lambda i: (0, 0)),
            pl.BlockSpec((O, E * r), lambda i: (0, 0)),
        ],
        out_specs=pl.BlockSpec((T, O), lambda i: (i, 0)),
        out_shape=jax.ShapeDtypeStruct((N, O), jnp.float32),
    )(xf, W, gate_W, A_flat, Bmat)
    return out.reshape(B, S, O)
